# baseline (speedup 1.0000x reference)
import functools

import jax
import jax.numpy as jnp
from jax import lax
from jax.experimental import pallas as pl
from jax.experimental.pallas import tpu as pltpu
from jax.experimental.pallas import tpu_sc as plsc

K_SEL = 5000
B = 4
HF, WF = 512, 512
HQ, WQ = 128, 128
NPB = HQ * WQ
NC, NS, L = 2, 16, 16
WPB = 8
QR_W = HQ // WPB
VALS = QR_W * WQ
NV = VALS // L
NBK = 18
ROWW = 128


def _mask_kernel(err_hbm, out_hbm, band, qbits, hist, outbuf, pub, rdbuf,
                 shared, dsem):
    c = lax.axis_index("c")
    s = lax.axis_index("s")
    bat = c * 2 + s // WPB
    w = s % WPB
    g0 = (s // WPB) * WPB

    lanes = lax.iota(jnp.int32, L)
    lanes4 = lanes * 4
    zero16 = jnp.zeros((L,), jnp.int32)
    ones16 = jnp.ones((L,), jnp.int32)

    nchunk = (QR_W // 2) * (WF // 128)

    def fire_dma(k, _):
        b8 = k >> 2
        t = k & 3
        pltpu.async_copy(
            err_hbm.at[bat, pl.ds(w * 4 * QR_W + b8 * 8, 8),
                       pl.ds(t * 128, 128)],
            band.at[pl.ds(k * 8, 8)], dsem)
        return 0

    lax.fori_loop(0, nchunk, fire_dma, 0)
    for j in range(NBK):
        hist[pl.ds(j * L, L)] = zero16

    def drain_dma(k, _):
        pltpu.make_async_copy(
            err_hbm.at[bat, pl.ds(0, 8), pl.ds(0, 128)],
            band.at[pl.ds(0, 8)], dsem).wait()
        return 0

    lax.fori_loop(0, nchunk, drain_dma, 0)

    def body_a(t, _):
        i = t >> 3
        vc = t & 7
        row1 = (i >> 1) * 32 + (vc >> 1) * 8 + ((i & 1) * 4 + 1)
        r1 = jnp.full((L,), row1, jnp.int32)
        r2 = r1 + 1
        c1 = ((vc & 1) * 64 + 1) + lanes4
        g = ((plsc.load_gather(band, [r1, c1])
              + plsc.load_gather(band, [r2, c1]))
             + (plsc.load_gather(band, [r1, c1 + 1])
                + plsc.load_gather(band, [r2, c1 + 1])))
        bits = plsc.bitcast(g * 0.25, jnp.int32)
        qbits[pl.ds(t * L, L)] = bits
        idx = ((bits >> 28) + 1) * L + lanes
        plsc.addupdate_scatter(hist, [idx], ones16)
        return 0

    lax.fori_loop(0, NV, body_a, 0)

    lanes16 = lanes * L + L

    def round_body(r, carry):
        lo, n_ge, eqb, ceq = carry
        tot = lax.fori_loop(
            0, L,
            lambda l, acc: acc + plsc.load_gather(hist, [lanes16 + l]),
            zero16)

        par = r & 1
        pub[pl.ds(0, L)] = tot
        pltpu.sync_copy(pub, shared.at[par, s])
        plsc.subcore_barrier()
        pltpu.sync_copy(shared.at[par, pl.ds(g0, WPB)], rdbuf)
        rows = [rdbuf[j, pl.ds(0, L)] for j in range(WPB)]
        gtot = rows[0]
        for j in range(1, WPB):
            gtot = gtot + rows[j]

        sfx = lax.rev(jnp.cumsum(lax.rev(gtot, (0,))), (0,))
        ge_k = (sfx + n_ge) >= K_SEL
        jstar = jnp.max(plsc.all_reduce_population_count(ge_k)) - 1
        above = jnp.sum(jnp.where(lanes == jstar + 1, sfx, 0))
        shift = 28 - 4 * r
        lo = lo + jnp.left_shift(jstar, shift)
        n_ge = n_ge + above
        sel_j = lanes == jstar
        ceq = jnp.sum(jnp.where(sel_j, tot, 0))
        eqb = jnp.int32(0)
        for j in range(WPB):
            vj = jnp.sum(jnp.where(sel_j, rows[j], 0))
            eqb = eqb + jnp.where(jnp.int32(j) < w, vj, 0)

        @pl.when(r < 7)
        def _():
            for j in range(NBK):
                hist[pl.ds(j * L, L)] = zero16
            nshift = shift - 4
            slo1 = (lo >> nshift) - 1

            def hist_body(v, _):
                bits = qbits[pl.ds(v * L, L)]
                bucket = jnp.clip((bits >> nshift) - slo1, 0, NBK - 1)
                plsc.addupdate_scatter(hist, [bucket * L + lanes], ones16)
                return 0

            lax.fori_loop(0, NV, hist_body, 0)

        return lo, n_ge, eqb, ceq

    thr, n_gt, eq_before, cnt_eq = lax.fori_loop(
        0, 8, round_body,
        (jnp.int32(0), jnp.int32(0), jnp.int32(0), jnp.int32(0)))

    take = jnp.clip(K_SEL - n_gt - eq_before, 0, cnt_eq)
    partial = (take > 0) & (take < cnt_eq)

    @pl.when(jnp.logical_not(partial))
    def _():
        any_eq = take > 0

        def out_simple(v, _):
            bits = qbits[pl.ds(v * L, L)]
            sel = (bits > thr) | ((bits == thr) & any_eq)
            outbuf[pl.ds(v * L, L)] = jnp.where(sel, 1.0, 0.0)
            return 0

        lax.fori_loop(0, NV, out_simple, 0)

    @pl.when(partial)
    def _():
        def out_ranked(v, running):
            bits = qbits[pl.ds(v * L, L)]
            gt = bits > thr
            eq = bits == thr
            eqi = jnp.where(eq, 1, 0)
            cum = plsc.cumsum(eqi)
            rank = (running + cum) - 1
            sel = gt | (eq & (rank < take))
            outbuf[pl.ds(v * L, L)] = jnp.where(sel, 1.0, 0.0)
            return running + jnp.sum(eqi)

        lax.fori_loop(0, NV, out_ranked, jnp.int32(0))

    pltpu.sync_copy(outbuf, out_hbm.at[pl.ds(bat * NPB + w * VALS, VALS)])


@jax.jit
def _refine_mask(err):
    run = functools.partial(
        pl.kernel,
        out_type=jax.ShapeDtypeStruct((B * NPB,), jnp.float32),
        mesh=plsc.VectorSubcoreMesh(
            core_axis_name="c", subcore_axis_name="s",
            num_cores=NC, num_subcores=NS),
        compiler_params=pltpu.CompilerParams(needs_layout_passes=False),
        scratch_types=[
            pltpu.VMEM((QR_W * 4 * WF // 128, 128), jnp.float32),
            pltpu.VMEM((VALS,), jnp.int32),
            pltpu.VMEM((NBK * L,), jnp.int32),
            pltpu.VMEM((VALS,), jnp.float32),
            pltpu.VMEM((ROWW,), jnp.int32),
            pltpu.VMEM((WPB, ROWW), jnp.int32),
            pltpu.VMEM_SHARED((2, NS, ROWW), jnp.int32),
            pltpu.SemaphoreType.DMA,
        ],
    )(_mask_kernel)
    return run(err)


def kernel(src, bck, alp, fgr, err, hid):
    flat = _refine_mask(err.reshape(B, HF, WF))
    return flat.reshape(B, 1, HQ, WQ)


# speedup vs baseline: 1.1724x; 1.1724x over previous
import functools

import jax
import jax.numpy as jnp
from jax import lax
from jax.experimental import pallas as pl
from jax.experimental.pallas import tpu as pltpu
from jax.experimental.pallas import tpu_sc as plsc

K_SEL = 5000
B = 4
HF, WF = 512, 512
HQ, WQ = 128, 128
NPB = HQ * WQ
NC, NS, L = 2, 16, 16
WPB = 8
QR_W = HQ // WPB
VALS = QR_W * WQ
NV = VALS // L
NBK = 18
ROWW = 128


def _mask_kernel(err_hbm, out_hbm, band, qbits, cand, hist, outbuf, pub,
                 rdbuf, shared, dsem):
    c = lax.axis_index("c")
    s = lax.axis_index("s")
    bat = c * 2 + s // WPB
    w = s % WPB
    g0 = (s // WPB) * WPB

    lanes = lax.iota(jnp.int32, L)
    lanes4 = lanes * 4
    zero16 = jnp.zeros((L,), jnp.int32)
    ones16 = jnp.ones((L,), jnp.int32)

    copies = [
        pltpu.async_copy(
            err_hbm.at[bat, pl.ds(w * 4 * QR_W + b8 * 8, 8),
                       pl.ds(t * 128, 128)],
            band.at[pl.ds((b8 * 4 + t) * 8, 8)], dsem)
        for b8 in range(QR_W // 2) for t in range(WF // 128)
    ]
    for j in range(NBK):
        hist[pl.ds(j * L, L)] = zero16
    for cp in copies:
        cp.wait()

    def body_a(t, _):
        i = t >> 3
        vc = t & 7
        row1 = (i >> 1) * 32 + (vc >> 1) * 8 + ((i & 1) * 4 + 1)
        r1 = jnp.full((L,), row1, jnp.int32)
        r2 = r1 + 1
        c1 = ((vc & 1) * 64 + 1) + lanes4
        g = ((plsc.load_gather(band, [r1, c1])
              + plsc.load_gather(band, [r2, c1]))
             + (plsc.load_gather(band, [r1, c1 + 1])
                + plsc.load_gather(band, [r2, c1 + 1])))
        bits = plsc.bitcast(g * 0.25, jnp.int32)
        qbits[pl.ds(t * L, L)] = bits
        cand[pl.ds(t * L, L)] = bits
        idx = ((bits >> 28) + 1) * L + lanes
        plsc.addupdate_scatter(hist, [idx], ones16)
        return 0

    lax.fori_loop(0, NV, body_a, 0)

    lanes16 = lanes * L + L

    def round_body(r, carry):
        lo, n_ge, eqb, ceq, m = carry
        tot = zero16
        for l in range(L):
            tot = tot + plsc.load_gather(hist, [lanes16 + l])

        par = r & 1
        pub[pl.ds(0, L)] = tot
        pltpu.sync_copy(pub, shared.at[par, s])
        plsc.subcore_barrier()
        pltpu.sync_copy(shared.at[par, pl.ds(g0, WPB)], rdbuf)
        rows = [rdbuf[j, pl.ds(0, L)] for j in range(WPB)]
        gtot = rows[0]
        for j in range(1, WPB):
            gtot = gtot + rows[j]

        sfx = lax.rev(jnp.cumsum(lax.rev(gtot, (0,))), (0,))
        ge_k = (sfx + n_ge) >= K_SEL
        jstar = jnp.max(plsc.all_reduce_population_count(ge_k)) - 1
        above = jnp.sum(jnp.where(lanes == jstar + 1, sfx, 0))
        shift = 28 - 4 * r
        lo = lo + jnp.left_shift(jstar, shift)
        n_ge = n_ge + above
        sel_j = lanes == jstar
        ceq = jnp.sum(jnp.where(sel_j, tot, 0))
        eqb = jnp.int32(0)
        for j in range(WPB):
            vj = jnp.sum(jnp.where(sel_j, rows[j], 0))
            eqb = eqb + jnp.where(jnp.int32(j) < w, vj, 0)

        nshift = shift - 4
        slo1 = (lo >> nshift) - 1

        def hist_body(v, off):
            valid = (v * L + lanes) < m
            bits = cand[pl.ds(v * L, L)]
            bucket = jnp.clip((bits >> nshift) - slo1, 0, NBK - 1)
            bucket = jnp.where(valid, bucket, 0)
            plsc.addupdate_scatter(hist, [bucket * L + lanes], ones16)
            inwin = valid & (bucket >= 1) & (bucket <= 16)
            plsc.store_compressed(cand.at[pl.ds(off, L)], bits, mask=inwin)
            return off + jnp.max(plsc.all_reduce_population_count(inwin))

        @pl.when(r < 7)
        def _():
            for j in range(NBK):
                hist[pl.ds(j * L, L)] = zero16

        m_next = lax.cond(
            r < 7,
            lambda: lax.fori_loop(0, (m + L - 1) >> 4, hist_body,
                                  jnp.int32(0)),
            lambda: m)

        return lo, n_ge, eqb, ceq, m_next

    thr, n_gt, eq_before, cnt_eq, _m = lax.fori_loop(
        0, 8, round_body,
        (jnp.int32(0), jnp.int32(0), jnp.int32(0), jnp.int32(0),
         jnp.int32(VALS)))

    take = jnp.clip(K_SEL - n_gt - eq_before, 0, cnt_eq)
    partial = (take > 0) & (take < cnt_eq)

    @pl.when(jnp.logical_not(partial))
    def _():
        any_eq = take > 0

        def out_simple(v, _):
            bits = qbits[pl.ds(v * L, L)]
            sel = (bits > thr) | ((bits == thr) & any_eq)
            outbuf[pl.ds(v * L, L)] = jnp.where(sel, 1.0, 0.0)
            return 0

        lax.fori_loop(0, NV, out_simple, 0)

    @pl.when(partial)
    def _():
        def out_ranked(v, running):
            bits = qbits[pl.ds(v * L, L)]
            gt = bits > thr
            eq = bits == thr
            eqi = jnp.where(eq, 1, 0)
            cum = plsc.cumsum(eqi)
            rank = (running + cum) - 1
            sel = gt | (eq & (rank < take))
            outbuf[pl.ds(v * L, L)] = jnp.where(sel, 1.0, 0.0)
            return running + jnp.sum(eqi)

        lax.fori_loop(0, NV, out_ranked, jnp.int32(0))

    pltpu.sync_copy(outbuf, out_hbm.at[pl.ds(bat * NPB + w * VALS, VALS)])


@jax.jit
def _refine_mask(err):
    run = functools.partial(
        pl.kernel,
        out_type=jax.ShapeDtypeStruct((B * NPB,), jnp.float32),
        mesh=plsc.VectorSubcoreMesh(
            core_axis_name="c", subcore_axis_name="s",
            num_cores=NC, num_subcores=NS),
        compiler_params=pltpu.CompilerParams(needs_layout_passes=False),
        scratch_types=[
            pltpu.VMEM((QR_W * 4 * WF // 128, 128), jnp.float32),
            pltpu.VMEM((VALS,), jnp.int32),
            pltpu.VMEM((VALS + L,), jnp.int32),
            pltpu.VMEM((NBK * L,), jnp.int32),
            pltpu.VMEM((VALS,), jnp.float32),
            pltpu.VMEM((ROWW,), jnp.int32),
            pltpu.VMEM((WPB, ROWW), jnp.int32),
            pltpu.VMEM_SHARED((2, NS, ROWW), jnp.int32),
            pltpu.SemaphoreType.DMA,
        ],
    )(_mask_kernel)
    return run(err)


def kernel(src, bck, alp, fgr, err, hid):
    flat = _refine_mask(err.reshape(B, HF, WF))
    return flat.reshape(B, 1, HQ, WQ)
